# Initial kernel scaffold; baseline (speedup 1.0000x reference)
import jax
import jax.numpy as jnp
from jax.experimental import pallas as pl


def kernel(token_seq, table):
    raise NotImplementedError("write your pallas kernel here")


# speedup vs baseline: 1.8650x; 1.8650x over previous
import functools

import jax
import jax.numpy as jnp
from jax import lax
from jax.experimental import pallas as pl
from jax.experimental.pallas import tpu as pltpu
from jax.experimental.pallas import tpu_sc as plsc

VOCAB = 1000000
EMBED_DIM = 128
BATCH = 4096
HIST_LEN = 200

NC = 2
NS = 16
NW = NC * NS

B = BATCH * HIST_LEN
ROWS_PER_W = B // NW
CHUNK = 128
NCH = ROWS_PER_W // CHUNK

_mesh = plsc.VectorSubcoreMesh(core_axis_name="c", subcore_axis_name="s")


@functools.partial(
    pl.kernel,
    out_type=jax.ShapeDtypeStruct((B, EMBED_DIM), jnp.float32),
    mesh=_mesh,
    scratch_types=[
        pltpu.VMEM((NCH, CHUNK), jnp.int32),
        pltpu.VMEM((CHUNK, EMBED_DIM), jnp.float32),
        pltpu.VMEM((CHUNK, EMBED_DIM), jnp.float32),
        pltpu.SemaphoreType.DMA,
        pltpu.SemaphoreType.DMA,
    ],
)
def _gather_kernel(table_hbm, idx_hbm, out_hbm, idx_v, rows0, rows1,
                   gsem0, gsem1):
    wid = lax.axis_index("s") * NC + lax.axis_index("c")
    base = wid * ROWS_PER_W

    pltpu.sync_copy(idx_hbm.at[wid], idx_v)

    pltpu.async_copy(table_hbm.at[idx_v.at[0]], rows0, gsem0)

    @pl.loop(0, NCH, step=2)
    def _(g):
        pltpu.async_copy(table_hbm.at[idx_v.at[g + 1]], rows1, gsem1)
        pltpu.make_async_copy(table_hbm.at[idx_v.at[g]], rows0, gsem0).wait()
        pltpu.sync_copy(rows0, out_hbm.at[pl.ds(base + g * CHUNK, CHUNK)])

        @pl.when(g + 2 < NCH)
        def _():
            pltpu.async_copy(table_hbm.at[idx_v.at[g + 2]], rows0, gsem0)

        pltpu.make_async_copy(table_hbm.at[idx_v.at[g + 1]], rows1, gsem1).wait()
        pltpu.sync_copy(rows1, out_hbm.at[pl.ds(base + (g + 1) * CHUNK, CHUNK)])


def kernel(token_seq, table):
    idx = token_seq.reshape(NW, NCH, CHUNK)
    out = _gather_kernel(table, idx)
    return out.reshape(BATCH, HIST_LEN, EMBED_DIM)
